# baseline (speedup 1.0000x reference)
import functools

import jax
import jax.numpy as jnp
from jax import lax
from jax.experimental import pallas as pl
from jax.experimental.pallas import tpu as pltpu
from jax.experimental.pallas import tpu_sc as plsc

EMBED = 64
DEMO = 16
H1 = 128
H2 = 64
NC, NS = 2, 16
NW = NC * NS
CHUNK = 256
CW = 8192
QW = CW // 4
LOG_CW = CW.bit_length() - 1
LOG_QW = QW.bit_length() - 1
BM = 2048


def _pack_trunc(a, b):
    ua = lax.bitcast_convert_type(a, jnp.uint32)
    ub = lax.bitcast_convert_type(b, jnp.uint32)
    word = (ua & jnp.uint32(0xFFFF0000)) | (ub >> 16)
    return lax.bitcast_convert_type(word, jnp.float32)


def _table_transpose(tu_T, ts_T, n_steps):
    out_rows = n_steps * QW

    def body(tu_ref, ts_ref, ou_ref, os_ref):
        k = lax.broadcasted_iota(jnp.int32, (EMBED, H1), 0)
        c = lax.broadcasted_iota(jnp.int32, (EMBED, H1), 1)
        e_lo = (c == k).astype(jnp.bfloat16)
        e_hi = (c == k + EMBED).astype(jnp.bfloat16)
        dn = (((0,), (0,)), ((), ()))
        for t_ref, o_ref in ((tu_ref, ou_ref), (ts_ref, os_ref)):
            t = t_ref[...].astype(jnp.bfloat16)
            a = lax.dot_general(t[:, :QW], e_lo, dn,
                                preferred_element_type=jnp.float32)
            b = lax.dot_general(t[:, QW:2 * QW], e_lo, dn,
                                preferred_element_type=jnp.float32)
            c2 = lax.dot_general(t[:, 2 * QW:3 * QW], e_hi, dn,
                                 preferred_element_type=jnp.float32)
            d = lax.dot_general(t[:, 3 * QW:], e_hi, dn,
                                preferred_element_type=jnp.float32)
            o_ref[...] = _pack_trunc(a + c2, b + d)

    return pl.pallas_call(
        body,
        grid=(n_steps,),
        in_specs=[
            pl.BlockSpec((EMBED, CW), lambda i: (0, i)),
            pl.BlockSpec((EMBED, CW), lambda i: (0, i)),
        ],
        out_specs=[
            pl.BlockSpec((QW, H1), lambda i: (i, 0)),
            pl.BlockSpec((QW, H1), lambda i: (i, 0)),
        ],
        out_shape=[
            jax.ShapeDtypeStruct((out_rows, H1), jnp.float32),
            jax.ShapeDtypeStruct((out_rows, H1), jnp.float32),
        ],
        compiler_params=pltpu.CompilerParams(
            dimension_semantics=("parallel",)),
    )(tu_T, ts_T)


def _sc_gather(user_idx, song_idx, pu, ps):
    batch = user_idx.shape[0]
    b_per_w = batch // NW
    n_steps = b_per_w // CHUNK
    mesh = plsc.VectorSubcoreMesh(core_axis_name="c", subcore_axis_name="s")
    out_type = (
        jax.ShapeDtypeStruct((batch, H1), jnp.float32),
        jax.ShapeDtypeStruct((batch, H1), jnp.float32),
    )

    @functools.partial(
        pl.kernel,
        mesh=mesh,
        out_type=out_type,
        scratch_types=[
            pltpu.VMEM((CHUNK,), jnp.int32),
            pltpu.VMEM((CHUNK,), jnp.int32),
            pltpu.VMEM((CHUNK, H1), jnp.float32),
            pltpu.VMEM((CHUNK, H1), jnp.float32),
            pltpu.SemaphoreType.DMA,
            pltpu.SemaphoreType.DMA,
        ],
    )
    def gather_kernel(pu_hbm, ps_hbm, ui_hbm, si_hbm, uo_hbm, so_hbm,
                      ui_v, si_v, ur_v, sr_v, sem_u, sem_s):
        wid = lax.axis_index("s") * NC + lax.axis_index("c")
        base = wid * b_per_w
        for h in range(n_steps):
            hb = base + h * CHUNK
            pltpu.sync_copy(ui_hbm.at[pl.ds(hb, CHUNK)], ui_v)
            pltpu.sync_copy(si_hbm.at[pl.ds(hb, CHUNK)], si_v)
            ui = ui_v[...]
            si = si_v[...]
            ui_v[...] = ((ui >> LOG_CW) << LOG_QW) | (ui & (QW - 1))
            si_v[...] = ((si >> LOG_CW) << LOG_QW) | (si & (QW - 1))
            cu = pltpu.async_copy(pu_hbm.at[ui_v], ur_v, sem_u)
            cs = pltpu.async_copy(ps_hbm.at[si_v], sr_v, sem_s)
            cu.wait()
            pltpu.sync_copy(ur_v, uo_hbm.at[pl.ds(hb, CHUNK)])
            cs.wait()
            pltpu.sync_copy(sr_v, so_hbm.at[pl.ds(hb, CHUNK)])

    return gather_kernel(pu, ps, user_idx, song_idx)


def _unpack_embed(packed, idx):
    quad = (idx >> LOG_QW) & 3
    u = lax.bitcast_convert_type(packed, jnp.uint32)
    sel = jnp.where((quad & 1) != 0, u << 16, u & jnp.uint32(0xFFFF0000))
    val = lax.bitcast_convert_type(sel, jnp.float32)
    lane = lax.broadcasted_iota(jnp.int32, packed.shape, 1)
    keep = (lane < EMBED) == ((quad >> 1) == 0)
    return jnp.where(keep, val, jnp.float32(0.0)).astype(jnp.bfloat16)


def _mlp_tail_body(pu_ref, ps_ref, uq_ref, sq_ref, d_ref, wdu_ref, wds_ref,
                   w1d_ref, b1_ref, w2_ref, b2_ref, w3_ref, b3_ref, o_ref):
    eu = _unpack_embed(pu_ref[...], uq_ref[...])
    es = _unpack_embed(ps_ref[...], sq_ref[...])
    h = lax.dot_general(eu, wdu_ref[...], (((1,), (0,)), ((), ())),
                        preferred_element_type=jnp.float32)
    h = h + lax.dot_general(es, wds_ref[...], (((1,), (0,)), ((), ())),
                            preferred_element_type=jnp.float32)
    d_val = d_ref[...]
    d_val = jnp.where(jnp.isnan(d_val), jnp.float32(0.0), d_val)
    h = h + jnp.dot(d_val, w1d_ref[...], preferred_element_type=jnp.float32)
    h = jnp.maximum(h + b1_ref[...], 0.0)
    h2 = jnp.dot(h, w2_ref[...], preferred_element_type=jnp.float32)
    h2 = jnp.maximum(h2 + b2_ref[...], 0.0)
    logit = jnp.dot(h2, w3_ref[...], preferred_element_type=jnp.float32)
    o_ref[...] = jax.nn.sigmoid(logit + b3_ref[...])


def _mlp_tail(pu_g, ps_g, u_idx, s_idx, demo, W1u_dup, W1s_dup, W1d, b1,
              W2, b2, W3, b3):
    batch = pu_g.shape[0]
    return pl.pallas_call(
        _mlp_tail_body,
        grid=(batch // BM,),
        in_specs=[
            pl.BlockSpec((BM, H1), lambda i: (i, 0)),
            pl.BlockSpec((BM, H1), lambda i: (i, 0)),
            pl.BlockSpec((BM, 1), lambda i: (i, 0)),
            pl.BlockSpec((BM, 1), lambda i: (i, 0)),
            pl.BlockSpec((BM, DEMO), lambda i: (i, 0)),
            pl.BlockSpec((H1, H1), lambda i: (0, 0)),
            pl.BlockSpec((H1, H1), lambda i: (0, 0)),
            pl.BlockSpec((DEMO, H1), lambda i: (0, 0)),
            pl.BlockSpec((1, H1), lambda i: (0, 0)),
            pl.BlockSpec((H1, H2), lambda i: (0, 0)),
            pl.BlockSpec((1, H2), lambda i: (0, 0)),
            pl.BlockSpec((H2, 1), lambda i: (0, 0)),
            pl.BlockSpec((1, 1), lambda i: (0, 0)),
        ],
        out_specs=pl.BlockSpec((BM, 1), lambda i: (i, 0)),
        out_shape=jax.ShapeDtypeStruct((batch, 1), jnp.float32),
        compiler_params=pltpu.CompilerParams(
            dimension_semantics=("parallel",)),
    )(pu_g, ps_g, u_idx.reshape(batch, 1), s_idx.reshape(batch, 1),
      demo, W1u_dup, W1s_dup, W1d, b1.reshape(1, H1), W2,
      b2.reshape(1, H2), W3, b3.reshape(1, 1))


def kernel(user_input, song_input, demographic_input, user_table, song_table,
           W1, b1, W2, b2, W3, b3):
    w1u = W1[:EMBED]
    w1s = W1[EMBED:2 * EMBED]
    w1d = W1[2 * EMBED:]
    w1u_dup = jnp.concatenate([w1u, w1u], axis=0).astype(jnp.bfloat16)
    w1s_dup = jnp.concatenate([w1s, w1s], axis=0).astype(jnp.bfloat16)
    n_rows = user_table.shape[0]
    n_steps = -(-n_rows // CW)
    pu, ps = _table_transpose(user_table.T, song_table.T, n_steps)
    pu_g, ps_g = _sc_gather(user_input, song_input, pu, ps)
    out = _mlp_tail(pu_g, ps_g, user_input, song_input, demographic_input,
                    w1u_dup, w1s_dup, w1d, b1, W2, b2, W3, b3)
    return out.reshape(user_input.shape[0])


# speedup vs baseline: 1.1064x; 1.1064x over previous
import functools

import jax
import jax.numpy as jnp
from jax import lax
from jax.experimental import pallas as pl
from jax.experimental.pallas import tpu as pltpu
from jax.experimental.pallas import tpu_sc as plsc

EMBED = 64
DEMO = 16
H1 = 128
H2 = 64
NC, NS = 2, 16
NW = NC * NS
CHUNK = 256
CW = 32768
QW = CW // 4
LOG_CW = CW.bit_length() - 1
LOG_QW = QW.bit_length() - 1
BM = 2048


def _pack_trunc(a, b):
    ua = lax.bitcast_convert_type(a, jnp.uint32)
    ub = lax.bitcast_convert_type(b, jnp.uint32)
    word = (ua & jnp.uint32(0xFFFF0000)) | (ub >> 16)
    return lax.bitcast_convert_type(word, jnp.float32)


def _table_transpose(tu_T, ts_T, n_steps):
    out_rows = n_steps * QW

    def body(tu_ref, ts_ref, ou_ref, os_ref):
        k = lax.broadcasted_iota(jnp.int32, (EMBED, H1), 0)
        c = lax.broadcasted_iota(jnp.int32, (EMBED, H1), 1)
        e_lo = (c == k).astype(jnp.bfloat16)
        e_hi = (c == k + EMBED).astype(jnp.bfloat16)
        dn = (((0,), (0,)), ((), ()))
        for t_ref, o_ref in ((tu_ref, ou_ref), (ts_ref, os_ref)):
            t = t_ref[...].astype(jnp.bfloat16)
            a = lax.dot_general(t[:, :QW], e_lo, dn,
                                preferred_element_type=jnp.float32)
            b = lax.dot_general(t[:, QW:2 * QW], e_lo, dn,
                                preferred_element_type=jnp.float32)
            c2 = lax.dot_general(t[:, 2 * QW:3 * QW], e_hi, dn,
                                 preferred_element_type=jnp.float32)
            d = lax.dot_general(t[:, 3 * QW:], e_hi, dn,
                                preferred_element_type=jnp.float32)
            o_ref[...] = _pack_trunc(a + c2, b + d)

    return pl.pallas_call(
        body,
        grid=(n_steps,),
        in_specs=[
            pl.BlockSpec((EMBED, CW), lambda i: (0, i)),
            pl.BlockSpec((EMBED, CW), lambda i: (0, i)),
        ],
        out_specs=[
            pl.BlockSpec((QW, H1), lambda i: (i, 0)),
            pl.BlockSpec((QW, H1), lambda i: (i, 0)),
        ],
        out_shape=[
            jax.ShapeDtypeStruct((out_rows, H1), jnp.float32),
            jax.ShapeDtypeStruct((out_rows, H1), jnp.float32),
        ],
        compiler_params=pltpu.CompilerParams(
            dimension_semantics=("parallel",)),
    )(tu_T, ts_T)


def _sc_gather(user_idx, song_idx, pu, ps):
    batch = user_idx.shape[0]
    b_per_w = batch // NW
    n_steps = b_per_w // CHUNK
    mesh = plsc.VectorSubcoreMesh(core_axis_name="c", subcore_axis_name="s")
    out_type = (
        jax.ShapeDtypeStruct((batch, H1), jnp.float32),
        jax.ShapeDtypeStruct((batch, H1), jnp.float32),
    )

    @functools.partial(
        pl.kernel,
        mesh=mesh,
        out_type=out_type,
        scratch_types=[
            pltpu.VMEM((CHUNK,), jnp.int32),
            pltpu.VMEM((CHUNK,), jnp.int32),
            pltpu.VMEM((CHUNK, H1), jnp.float32),
            pltpu.VMEM((CHUNK, H1), jnp.float32),
            pltpu.SemaphoreType.DMA,
            pltpu.SemaphoreType.DMA,
        ],
    )
    def gather_kernel(pu_hbm, ps_hbm, ui_hbm, si_hbm, uo_hbm, so_hbm,
                      ui_v, si_v, ur_v, sr_v, sem_u, sem_s):
        wid = lax.axis_index("s") * NC + lax.axis_index("c")
        base = wid * b_per_w
        for h in range(n_steps):
            hb = base + h * CHUNK
            pltpu.sync_copy(ui_hbm.at[pl.ds(hb, CHUNK)], ui_v)
            pltpu.sync_copy(si_hbm.at[pl.ds(hb, CHUNK)], si_v)
            ui = ui_v[...]
            si = si_v[...]
            ui_v[...] = ((ui >> LOG_CW) << LOG_QW) | (ui & (QW - 1))
            si_v[...] = ((si >> LOG_CW) << LOG_QW) | (si & (QW - 1))
            cu = pltpu.async_copy(pu_hbm.at[ui_v], ur_v, sem_u)
            cs = pltpu.async_copy(ps_hbm.at[si_v], sr_v, sem_s)
            cu.wait()
            pltpu.sync_copy(ur_v, uo_hbm.at[pl.ds(hb, CHUNK)])
            cs.wait()
            pltpu.sync_copy(sr_v, so_hbm.at[pl.ds(hb, CHUNK)])

    return gather_kernel(pu, ps, user_idx, song_idx)


def _unpack_embed(packed, idx):
    quad = (idx >> LOG_QW) & 3
    u = lax.bitcast_convert_type(packed, jnp.uint32)
    shift = ((quad & 1) << 4).astype(jnp.uint32)
    sel = (u << shift) & jnp.uint32(0xFFFF0000)
    val = lax.bitcast_convert_type(sel, jnp.float32)
    lane = lax.broadcasted_iota(jnp.int32, packed.shape, 1)
    keep = (lane < EMBED) == (quad < 2)
    return jnp.where(keep, val, jnp.float32(0.0)).astype(jnp.bfloat16)


def _mlp_tail_body(pu_ref, ps_ref, uq_ref, sq_ref, d_ref, wdu_ref, wds_ref,
                   w1d_ref, b1_ref, w2_ref, b2_ref, w3_ref, b3_ref, o_ref):
    eu = _unpack_embed(pu_ref[...], uq_ref[...])
    es = _unpack_embed(ps_ref[...], sq_ref[...])
    h = lax.dot_general(eu, wdu_ref[...], (((1,), (0,)), ((), ())),
                        preferred_element_type=jnp.float32)
    h = h + lax.dot_general(es, wds_ref[...], (((1,), (0,)), ((), ())),
                            preferred_element_type=jnp.float32)
    d_val = d_ref[...]
    d_val = jnp.where(jnp.isnan(d_val), jnp.float32(0.0), d_val)
    h = h + jnp.dot(d_val, w1d_ref[...], preferred_element_type=jnp.float32)
    h = jnp.maximum(h + b1_ref[...], 0.0)
    h2 = jnp.dot(h, w2_ref[...], preferred_element_type=jnp.float32)
    h2 = jnp.maximum(h2 + b2_ref[...], 0.0)
    logit = jnp.dot(h2, w3_ref[...], preferred_element_type=jnp.float32)
    o_ref[...] = jax.nn.sigmoid(logit + b3_ref[...])


def _mlp_tail(pu_g, ps_g, u_idx, s_idx, demo, W1u_dup, W1s_dup, W1d, b1,
              W2, b2, W3, b3):
    batch = pu_g.shape[0]
    return pl.pallas_call(
        _mlp_tail_body,
        grid=(batch // BM,),
        in_specs=[
            pl.BlockSpec((BM, H1), lambda i: (i, 0)),
            pl.BlockSpec((BM, H1), lambda i: (i, 0)),
            pl.BlockSpec((BM, 1), lambda i: (i, 0)),
            pl.BlockSpec((BM, 1), lambda i: (i, 0)),
            pl.BlockSpec((BM, DEMO), lambda i: (i, 0)),
            pl.BlockSpec((H1, H1), lambda i: (0, 0)),
            pl.BlockSpec((H1, H1), lambda i: (0, 0)),
            pl.BlockSpec((DEMO, H1), lambda i: (0, 0)),
            pl.BlockSpec((1, H1), lambda i: (0, 0)),
            pl.BlockSpec((H1, H2), lambda i: (0, 0)),
            pl.BlockSpec((1, H2), lambda i: (0, 0)),
            pl.BlockSpec((H2, 1), lambda i: (0, 0)),
            pl.BlockSpec((1, 1), lambda i: (0, 0)),
        ],
        out_specs=pl.BlockSpec((BM, 1), lambda i: (i, 0)),
        out_shape=jax.ShapeDtypeStruct((batch, 1), jnp.float32),
        compiler_params=pltpu.CompilerParams(
            dimension_semantics=("parallel",)),
    )(pu_g, ps_g, u_idx.reshape(batch, 1), s_idx.reshape(batch, 1),
      demo, W1u_dup, W1s_dup, W1d, b1.reshape(1, H1), W2,
      b2.reshape(1, H2), W3, b3.reshape(1, 1))


def kernel(user_input, song_input, demographic_input, user_table, song_table,
           W1, b1, W2, b2, W3, b3):
    w1u = W1[:EMBED]
    w1s = W1[EMBED:2 * EMBED]
    w1d = W1[2 * EMBED:]
    w1u_dup = jnp.concatenate([w1u, w1u], axis=0).astype(jnp.bfloat16)
    w1s_dup = jnp.concatenate([w1s, w1s], axis=0).astype(jnp.bfloat16)
    n_rows = user_table.shape[0]
    n_steps = -(-n_rows // CW)
    pu, ps = _table_transpose(user_table.T, song_table.T, n_steps)
    pu_g, ps_g = _sc_gather(user_input, song_input, pu, ps)
    out = _mlp_tail(pu_g, ps_g, user_input, song_input, demographic_input,
                    w1u_dup, w1s_dup, w1d, b1, W2, b2, W3, b3)
    return out.reshape(user_input.shape[0])
